# baseline (speedup 1.0000x reference)
import functools

import jax
import jax.numpy as jnp
from jax import lax
from jax.experimental import pallas as pl
from jax.experimental.pallas import tpu as pltpu
from jax.experimental.pallas import tpu_sc as plsc

_B = 4096
_D = 64
_T = 200000
_OFF = 100000
_CHUNK = 2048
_NBLK = (_T + _CHUNK - 1) // _CHUNK
_GROWS = _NBLK * _CHUNK * _D // 16
_NC, _NS = 2, 16
_NW = _NC * _NS
_BPW = _B // _NW


def _transpose_body(in_ref, out_ref):
    eye = jnp.asarray(
        lax.broadcasted_iota(jnp.int32, (_D, _D), 0)
        == lax.broadcasted_iota(jnp.int32, (_D, _D), 1), jnp.float32)
    t = lax.dot_general(in_ref[...], eye, (((0,), (0,)), ((), ())),
                        preferred_element_type=jnp.float32)
    out_ref[:, 0:_D] = t[0:_CHUNK // 2]
    out_ref[:, _D:2 * _D] = t[_CHUNK // 2:_CHUNK]


_transposer = pl.pallas_call(
    _transpose_body,
    grid=(_NBLK,),
    in_specs=[pl.BlockSpec((_D, _CHUNK), lambda j: (0, j))],
    out_specs=pl.BlockSpec((_CHUNK // 2, 128), lambda j: (j, 0)),
    out_shape=jax.ShapeDtypeStruct((_NBLK * _CHUNK // 2, 128), jnp.float32),
)


@functools.lru_cache(maxsize=None)
def _make_sc_gather():
    mesh = plsc.VectorSubcoreMesh(core_axis_name="c", subcore_axis_name="s")

    @functools.partial(
        pl.kernel,
        mesh=mesh,
        compiler_params=pltpu.CompilerParams(use_tc_tiling_on_sc=False,
                                             needs_layout_passes=False),
        out_type=[
            jax.ShapeDtypeStruct((_B * 8, 16), jnp.float32),
            jax.ShapeDtypeStruct((_B * 8, 16), jnp.float32),
        ],
        scratch_types=[
            pltpu.VMEM((_BPW,), jnp.int32),
            pltpu.VMEM((_BPW,), jnp.int32),
            pltpu.VMEM((8, 128), jnp.int32),
            pltpu.VMEM((8, 128), jnp.int32),
            pltpu.VMEM((_BPW,), jnp.int32),
            pltpu.VMEM((_BPW,), jnp.int32),
            pltpu.VMEM((_BPW * 8, 16), jnp.float32),
            pltpu.VMEM((_BPW * 8, 16), jnp.float32),
            pltpu.VMEM((_BPW, 16), jnp.float32),
            pltpu.VMEM((_BPW, 16), jnp.float32),
            pltpu.SemaphoreType.DMA,
            pltpu.SemaphoreType.DMA,
        ],
    )
    def _sc_gather(x1_hbm, x2_hbm, gt_hbm, lin_hbm,
                   r1_out, r2_out,
                   idx1_v, idx2_v, g1_v, g2_v, lrow1_v, lrow2_v,
                   rows1_v, rows2_v, lbuf1_v, lbuf2_v,
                   sem1, sem2):
        wid = lax.axis_index("s") * _NC + lax.axis_index("c")
        base = wid * _BPW
        pltpu.sync_copy(x1_hbm.at[pl.ds(base, _BPW)], idx1_v)
        pltpu.sync_copy(x2_hbm.at[pl.ds(base, _BPW)], idx2_v)
        lane = lax.iota(jnp.int32, 16)
        for i in range(_BPW // 16):
            sl = pl.ds(i * 16, 16)
            idx2_v[sl] = idx2_v[sl] + _OFF

        for idx_v, g_v, lrow_v, rows_v, lbuf_v, r_out, sem in (
            (idx1_v, g1_v, lrow1_v, rows1_v, lbuf1_v, r1_out, sem1),
            (idx2_v, g2_v, lrow2_v, rows2_v, lbuf2_v, r2_out, sem2),
        ):
            for j in range(8):
                sl = pl.ds(j * 16, 16)
                v = idx_v[sl]
                jb = jnp.right_shift(v, 11)
                pos = v & 2047
                g0 = (jnp.left_shift(jb, 13)
                      + jnp.left_shift(pos & 1023, 3)
                      + jnp.left_shift(jnp.right_shift(pos, 10), 2))
                row = lane * 0 + j
                for k in range(4):
                    plsc.store_scatter(g_v, [row, lane * 8 + k], g0 + k)
                for k in (4, 5, 6, 7):
                    plsc.store_scatter(g_v, [row, lane * 8 + k], g0)
                lrow_v[sl] = jnp.right_shift(v, 4)
            cps = [
                pltpu.async_copy(gt_hbm.at[g_v.at[j]],
                                 rows_v.at[pl.ds(j * 128, 128)], sem)
                for j in range(8)
            ]
            cpl = pltpu.async_copy(lin_hbm.at[lrow_v], lbuf_v, sem)
            for cp in cps:
                cp.wait()
            cpl.wait()
            for j in range(8):
                sl = pl.ds(j * 16, 16)
                samp = lane + (j * 16)
                vals = plsc.load_gather(lbuf_v, [samp, idx_v[sl] & 15])
                plsc.store_scatter(rows_v, [samp * 8 + 4, lane * 0], vals)
            pltpu.sync_copy(rows_v, r_out.at[pl.ds(base * 8, _BPW * 8)])

    return _sc_gather


def _dense_body(r1_ref, r2_ref, lin_b_ref,
                w1_ref, b1_ref, g1_ref, be1_ref,
                w2_ref, b2_ref, g2_ref, be2_ref,
                w3_ref, b3_ref, out_ref):
    e1 = r1_ref[:, 0:_D]
    e2 = r2_ref[:, 0:_D]

    s = e1 + e2
    fm = 0.5 * jnp.sum(s * s - e1 * e1 - e2 * e2, axis=1, keepdims=True)

    lin = r1_ref[:, _D:_D + 1] + r2_ref[:, _D:_D + 1] + lin_b_ref[...]

    h = (
        jnp.dot(e1, w1_ref[0:_D, :], preferred_element_type=jnp.float32)
        + jnp.dot(e2, w1_ref[_D:2 * _D, :], preferred_element_type=jnp.float32)
        + b1_ref[...]
    )
    m = jnp.mean(h, axis=0, keepdims=True)
    hc = h - m
    v = jnp.mean(hc * hc, axis=0, keepdims=True)
    h = jnp.maximum(hc * lax.rsqrt(v + 1e-5) * g1_ref[...] + be1_ref[...], 0.0)

    h = jnp.dot(h, w2_ref[...], preferred_element_type=jnp.float32) + b2_ref[...]
    m = jnp.mean(h, axis=0, keepdims=True)
    hc = h - m
    v = jnp.mean(hc * hc, axis=0, keepdims=True)
    h = jnp.maximum(hc * lax.rsqrt(v + 1e-5) * g2_ref[...] + be2_ref[...], 0.0)

    o = jnp.dot(h, w3_ref[...], preferred_element_type=jnp.float32) + b3_ref[...]
    z = lin + fm + o
    out_ref[...] = 1.0 / (1.0 + jnp.exp(-z))


_dense = pl.pallas_call(
    _dense_body,
    out_shape=jax.ShapeDtypeStruct((_B, 1), jnp.float32),
)


def kernel(x1, x2, emb_table, lin_w, lin_b,
           W1, b1, g1, be1, W2, b2, g2, be2, W3, b3):
    gt = _transposer(emb_table.T)
    r1, r2 = _make_sc_gather()(x1, x2, gt.reshape(_GROWS, 16),
                               lin_w.reshape(-1, 16))
    out = _dense(
        r1.reshape(_B, 128), r2.reshape(_B, 128), lin_b.reshape(1, 1),
        W1, b1.reshape(1, -1), g1.reshape(1, -1), be1.reshape(1, -1),
        W2, b2.reshape(1, -1), g2.reshape(1, -1), be2.reshape(1, -1),
        W3, b3.reshape(1, 1),
    )
    return out.reshape(_B)


# speedup vs baseline: 1.2362x; 1.2362x over previous
import functools

import jax
import jax.numpy as jnp
from jax import lax
from jax.experimental import pallas as pl
from jax.experimental.pallas import tpu as pltpu
from jax.experimental.pallas import tpu_sc as plsc

_B = 4096
_D = 64
_T = 200000
_OFF = 100000
_CHUNK = 4096
_NBLK = (_T + _CHUNK - 1) // _CHUNK
_GROWS = _NBLK * _CHUNK * _D // 16
_NC, _NS = 2, 16
_NW = _NC * _NS
_BPW = _B // _NW


def _transpose_body(in_ref, out_ref):
    eye = jnp.asarray(
        lax.broadcasted_iota(jnp.int32, (_D, _D), 0)
        == lax.broadcasted_iota(jnp.int32, (_D, _D), 1), jnp.float32)
    t = lax.dot_general(in_ref[...], eye, (((0,), (0,)), ((), ())),
                        preferred_element_type=jnp.float32)
    out_ref[:, 0:_D] = t[0:_CHUNK // 2]
    out_ref[:, _D:2 * _D] = t[_CHUNK // 2:_CHUNK]


_transposer = pl.pallas_call(
    _transpose_body,
    grid=(_NBLK,),
    in_specs=[pl.BlockSpec((_D, _CHUNK), lambda j: (0, j))],
    out_specs=pl.BlockSpec((_CHUNK // 2, 128), lambda j: (j, 0)),
    out_shape=jax.ShapeDtypeStruct((_NBLK * _CHUNK // 2, 128), jnp.float32),
)


@functools.lru_cache(maxsize=None)
def _make_sc_gather():
    mesh = plsc.VectorSubcoreMesh(core_axis_name="c", subcore_axis_name="s")

    @functools.partial(
        pl.kernel,
        mesh=mesh,
        compiler_params=pltpu.CompilerParams(use_tc_tiling_on_sc=False,
                                             needs_layout_passes=False),
        out_type=[
            jax.ShapeDtypeStruct((_B * 8, 16), jnp.float32),
            jax.ShapeDtypeStruct((_B * 8, 16), jnp.float32),
        ],
        scratch_types=[
            pltpu.VMEM((_BPW,), jnp.int32),
            pltpu.VMEM((_BPW,), jnp.int32),
            pltpu.VMEM((8, 128), jnp.int32),
            pltpu.VMEM((8, 128), jnp.int32),
            pltpu.VMEM((_BPW,), jnp.int32),
            pltpu.VMEM((_BPW,), jnp.int32),
            pltpu.VMEM((_BPW * 8, 16), jnp.float32),
            pltpu.VMEM((_BPW * 8, 16), jnp.float32),
            pltpu.VMEM((_BPW, 16), jnp.float32),
            pltpu.VMEM((_BPW, 16), jnp.float32),
            pltpu.SemaphoreType.DMA,
            pltpu.SemaphoreType.DMA,
        ],
    )
    def _sc_gather(x1_hbm, x2_hbm, gt_hbm, lin_hbm,
                   r1_out, r2_out,
                   idx1_v, idx2_v, g1_v, g2_v, lrow1_v, lrow2_v,
                   rows1_v, rows2_v, lbuf1_v, lbuf2_v,
                   sem1, sem2):
        wid = lax.axis_index("s") * _NC + lax.axis_index("c")
        base = wid * _BPW
        pltpu.sync_copy(x1_hbm.at[pl.ds(base, _BPW)], idx1_v)
        pltpu.sync_copy(x2_hbm.at[pl.ds(base, _BPW)], idx2_v)
        lane = lax.iota(jnp.int32, 16)
        for i in range(_BPW // 16):
            sl = pl.ds(i * 16, 16)
            idx2_v[sl] = idx2_v[sl] + _OFF

        for idx_v, g_v, lrow_v, rows_v, lbuf_v, r_out, sem in (
            (idx1_v, g1_v, lrow1_v, rows1_v, lbuf1_v, r1_out, sem1),
            (idx2_v, g2_v, lrow2_v, rows2_v, lbuf2_v, r2_out, sem2),
        ):
            for j in range(8):
                sl = pl.ds(j * 16, 16)
                v = idx_v[sl]
                jb = jnp.right_shift(v, 12)
                pos = v & (_CHUNK - 1)
                g0 = (jnp.left_shift(jb, 14)
                      + jnp.left_shift(pos & (_CHUNK // 2 - 1), 3)
                      + jnp.left_shift(jnp.right_shift(pos, 11), 2))
                row = lane * 0 + j
                for k in range(4):
                    plsc.store_scatter(g_v, [row, lane * 8 + k], g0 + k)
                for k in (4, 5, 6, 7):
                    plsc.store_scatter(g_v, [row, lane * 8 + k], g0)
                lrow_v[sl] = jnp.right_shift(v, 4)
            cps = [
                pltpu.async_copy(gt_hbm.at[g_v.at[j]],
                                 rows_v.at[pl.ds(j * 128, 128)], sem)
                for j in range(8)
            ]
            cpl = pltpu.async_copy(lin_hbm.at[lrow_v], lbuf_v, sem)
            for cp in cps:
                cp.wait()
            cpl.wait()
            for j in range(8):
                sl = pl.ds(j * 16, 16)
                samp = lane + (j * 16)
                vals = plsc.load_gather(lbuf_v, [samp, idx_v[sl] & 15])
                plsc.store_scatter(rows_v, [samp * 8 + 4, lane * 0], vals)
            pltpu.sync_copy(rows_v, r_out.at[pl.ds(base * 8, _BPW * 8)])

    return _sc_gather


def _dense_body(r1_ref, r2_ref, lin_b_ref,
                w1_ref, b1_ref, g1_ref, be1_ref,
                w2_ref, b2_ref, g2_ref, be2_ref,
                w3_ref, b3_ref, out_ref):
    e1 = r1_ref[:, 0:_D]
    e2 = r2_ref[:, 0:_D]

    s = e1 + e2
    fm = 0.5 * jnp.sum(s * s - e1 * e1 - e2 * e2, axis=1, keepdims=True)

    lin = r1_ref[:, _D:_D + 1] + r2_ref[:, _D:_D + 1] + lin_b_ref[...]

    h = (
        jnp.dot(e1, w1_ref[0:_D, :], preferred_element_type=jnp.float32)
        + jnp.dot(e2, w1_ref[_D:2 * _D, :], preferred_element_type=jnp.float32)
        + b1_ref[...]
    )
    m = jnp.mean(h, axis=0, keepdims=True)
    hc = h - m
    v = jnp.mean(hc * hc, axis=0, keepdims=True)
    h = jnp.maximum(hc * lax.rsqrt(v + 1e-5) * g1_ref[...] + be1_ref[...], 0.0)

    h = jnp.dot(h, w2_ref[...], preferred_element_type=jnp.float32) + b2_ref[...]
    m = jnp.mean(h, axis=0, keepdims=True)
    hc = h - m
    v = jnp.mean(hc * hc, axis=0, keepdims=True)
    h = jnp.maximum(hc * lax.rsqrt(v + 1e-5) * g2_ref[...] + be2_ref[...], 0.0)

    o = jnp.dot(h, w3_ref[...], preferred_element_type=jnp.float32) + b3_ref[...]
    z = lin + fm + o
    out_ref[...] = 1.0 / (1.0 + jnp.exp(-z))


_dense = pl.pallas_call(
    _dense_body,
    out_shape=jax.ShapeDtypeStruct((_B, 1), jnp.float32),
)


def kernel(x1, x2, emb_table, lin_w, lin_b,
           W1, b1, g1, be1, W2, b2, g2, be2, W3, b3):
    gt = _transposer(emb_table.T)
    r1, r2 = _make_sc_gather()(x1, x2, gt.reshape(_GROWS, 16),
                               lin_w.reshape(-1, 16))
    out = _dense(
        r1.reshape(_B, 128), r2.reshape(_B, 128), lin_b.reshape(1, 1),
        W1, b1.reshape(1, -1), g1.reshape(1, -1), be1.reshape(1, -1),
        W2, b2.reshape(1, -1), g2.reshape(1, -1), be2.reshape(1, -1),
        W3, b3.reshape(1, 1),
    )
    return out.reshape(_B)


# speedup vs baseline: 1.3918x; 1.1259x over previous
import functools

import jax
import jax.numpy as jnp
from jax import lax
from jax.experimental import pallas as pl
from jax.experimental.pallas import tpu as pltpu
from jax.experimental.pallas import tpu_sc as plsc

_B = 4096
_D = 64
_T = 200000
_OFF = 100000
_CHUNK = 8192
_LOG2C = 13
_NBLK = (_T + _CHUNK - 1) // _CHUNK
_GROWS = _NBLK * _CHUNK * _D // 16
_NC, _NS = 2, 16
_NW = _NC * _NS
_BPW = _B // _NW


def _transpose_body(in_ref, out_ref):
    eye = jnp.asarray(
        lax.broadcasted_iota(jnp.int32, (_D, _D), 0)
        == lax.broadcasted_iota(jnp.int32, (_D, _D), 1), jnp.float32)
    t = lax.dot_general(in_ref[...], eye, (((0,), (0,)), ((), ())),
                        preferred_element_type=jnp.float32)
    out_ref[:, 0:_D] = t[0:_CHUNK // 2]
    out_ref[:, _D:2 * _D] = t[_CHUNK // 2:_CHUNK]


_transposer = pl.pallas_call(
    _transpose_body,
    grid=(_NBLK,),
    in_specs=[pl.BlockSpec((_D, _CHUNK), lambda j: (0, j))],
    out_specs=pl.BlockSpec((_CHUNK // 2, 128), lambda j: (j, 0)),
    out_shape=jax.ShapeDtypeStruct((_NBLK * _CHUNK // 2, 128), jnp.float32),
)


@functools.lru_cache(maxsize=None)
def _make_sc_gather():
    mesh = plsc.VectorSubcoreMesh(core_axis_name="c", subcore_axis_name="s")

    @functools.partial(
        pl.kernel,
        mesh=mesh,
        compiler_params=pltpu.CompilerParams(use_tc_tiling_on_sc=False,
                                             needs_layout_passes=False),
        out_type=[
            jax.ShapeDtypeStruct((_B * 8, 16), jnp.float32),
            jax.ShapeDtypeStruct((_B * 8, 16), jnp.float32),
        ],
        scratch_types=[
            pltpu.VMEM((_BPW,), jnp.int32),
            pltpu.VMEM((_BPW,), jnp.int32),
            pltpu.VMEM((8, 128), jnp.int32),
            pltpu.VMEM((8, 128), jnp.int32),
            pltpu.VMEM((_BPW,), jnp.int32),
            pltpu.VMEM((_BPW,), jnp.int32),
            pltpu.VMEM((_BPW * 8, 16), jnp.float32),
            pltpu.VMEM((_BPW * 8, 16), jnp.float32),
            pltpu.VMEM((_BPW, 16), jnp.float32),
            pltpu.VMEM((_BPW, 16), jnp.float32),
            pltpu.SemaphoreType.DMA,
            pltpu.SemaphoreType.DMA,
        ],
    )
    def _sc_gather(x1_hbm, x2_hbm, gt_hbm, lin_hbm,
                   r1_out, r2_out,
                   idx1_v, idx2_v, g1_v, g2_v, lrow1_v, lrow2_v,
                   rows1_v, rows2_v, lbuf1_v, lbuf2_v,
                   sem1, sem2):
        wid = lax.axis_index("s") * _NC + lax.axis_index("c")
        base = wid * _BPW
        pltpu.sync_copy(x1_hbm.at[pl.ds(base, _BPW)], idx1_v)
        pltpu.sync_copy(x2_hbm.at[pl.ds(base, _BPW)], idx2_v)
        lane = lax.iota(jnp.int32, 16)
        for i in range(_BPW // 16):
            sl = pl.ds(i * 16, 16)
            idx2_v[sl] = idx2_v[sl] + _OFF

        for idx_v, g_v, lrow_v, rows_v, lbuf_v, r_out, sem in (
            (idx1_v, g1_v, lrow1_v, rows1_v, lbuf1_v, r1_out, sem1),
            (idx2_v, g2_v, lrow2_v, rows2_v, lbuf2_v, r2_out, sem2),
        ):
            for j in range(8):
                sl = pl.ds(j * 16, 16)
                v = idx_v[sl]
                jb = jnp.right_shift(v, _LOG2C)
                pos = v & (_CHUNK - 1)
                g0 = (jnp.left_shift(jb, _LOG2C + 2)
                      + jnp.left_shift(pos & (_CHUNK // 2 - 1), 3)
                      + jnp.left_shift(jnp.right_shift(pos, _LOG2C - 1), 2))
                row = lane * 0 + j
                for k in range(4):
                    plsc.store_scatter(g_v, [row, lane * 8 + k], g0 + k)
                for k in (4, 5, 6, 7):
                    plsc.store_scatter(g_v, [row, lane * 8 + k], g0)
                lrow_v[sl] = jnp.right_shift(v, 4)
            cps = [
                pltpu.async_copy(gt_hbm.at[g_v.at[j]],
                                 rows_v.at[pl.ds(j * 128, 128)], sem)
                for j in range(8)
            ]
            cpl = pltpu.async_copy(lin_hbm.at[lrow_v], lbuf_v, sem)
            for cp in cps:
                cp.wait()
            cpl.wait()
            for j in range(8):
                sl = pl.ds(j * 16, 16)
                samp = lane + (j * 16)
                vals = plsc.load_gather(lbuf_v, [samp, idx_v[sl] & 15])
                plsc.store_scatter(rows_v, [samp * 8 + 4, lane * 0], vals)
            pltpu.sync_copy(rows_v, r_out.at[pl.ds(base * 8, _BPW * 8)])

    return _sc_gather


def _dense_body(r1_ref, r2_ref, lin_b_ref,
                w1_ref, b1_ref, g1_ref, be1_ref,
                w2_ref, b2_ref, g2_ref, be2_ref,
                w3_ref, b3_ref, out_ref):
    e1 = r1_ref[:, 0:_D]
    e2 = r2_ref[:, 0:_D]

    s = e1 + e2
    fm = 0.5 * jnp.sum(s * s - e1 * e1 - e2 * e2, axis=1, keepdims=True)

    lin = r1_ref[:, _D:_D + 1] + r2_ref[:, _D:_D + 1] + lin_b_ref[...]

    h = (
        jnp.dot(e1, w1_ref[0:_D, :], preferred_element_type=jnp.float32)
        + jnp.dot(e2, w1_ref[_D:2 * _D, :], preferred_element_type=jnp.float32)
        + b1_ref[...]
    )
    m = jnp.mean(h, axis=0, keepdims=True)
    hc = h - m
    v = jnp.mean(hc * hc, axis=0, keepdims=True)
    h = jnp.maximum(hc * lax.rsqrt(v + 1e-5) * g1_ref[...] + be1_ref[...], 0.0)

    h = jnp.dot(h, w2_ref[...], preferred_element_type=jnp.float32) + b2_ref[...]
    m = jnp.mean(h, axis=0, keepdims=True)
    hc = h - m
    v = jnp.mean(hc * hc, axis=0, keepdims=True)
    h = jnp.maximum(hc * lax.rsqrt(v + 1e-5) * g2_ref[...] + be2_ref[...], 0.0)

    o = jnp.dot(h, w3_ref[...], preferred_element_type=jnp.float32) + b3_ref[...]
    z = lin + fm + o
    out_ref[...] = 1.0 / (1.0 + jnp.exp(-z))


_dense = pl.pallas_call(
    _dense_body,
    out_shape=jax.ShapeDtypeStruct((_B, 1), jnp.float32),
)


def kernel(x1, x2, emb_table, lin_w, lin_b,
           W1, b1, g1, be1, W2, b2, g2, be2, W3, b3):
    gt = _transposer(emb_table.T)
    r1, r2 = _make_sc_gather()(x1, x2, gt.reshape(_GROWS, 16),
                               lin_w.reshape(-1, 16))
    out = _dense(
        r1.reshape(_B, 128), r2.reshape(_B, 128), lin_b.reshape(1, 1),
        W1, b1.reshape(1, -1), g1.reshape(1, -1), be1.reshape(1, -1),
        W2, b2.reshape(1, -1), g2.reshape(1, -1), be2.reshape(1, -1),
        W3, b3.reshape(1, 1),
    )
    return out.reshape(_B)


# speedup vs baseline: 1.4587x; 1.0481x over previous
import functools

import jax
import jax.numpy as jnp
from jax import lax
from jax.experimental import pallas as pl
from jax.experimental.pallas import tpu as pltpu
from jax.experimental.pallas import tpu_sc as plsc

_B = 4096
_D = 64
_T = 200000
_OFF = 100000
_CHUNK = 16384
_LOG2C = 14
_NBLK = (_T + _CHUNK - 1) // _CHUNK
_GROWS = _NBLK * _CHUNK * _D // 16
_NC, _NS = 2, 16
_NW = _NC * _NS
_BPW = _B // _NW


def _transpose_body(in_ref, out_ref):
    eye = jnp.asarray(
        lax.broadcasted_iota(jnp.int32, (_D, _D), 0)
        == lax.broadcasted_iota(jnp.int32, (_D, _D), 1), jnp.float32)
    t = lax.dot_general(in_ref[...], eye, (((0,), (0,)), ((), ())),
                        preferred_element_type=jnp.float32)
    out_ref[:, 0:_D] = t[0:_CHUNK // 2]
    out_ref[:, _D:2 * _D] = t[_CHUNK // 2:_CHUNK]


_transposer = pl.pallas_call(
    _transpose_body,
    grid=(_NBLK,),
    in_specs=[pl.BlockSpec((_D, _CHUNK), lambda j: (0, j))],
    out_specs=pl.BlockSpec((_CHUNK // 2, 128), lambda j: (j, 0)),
    out_shape=jax.ShapeDtypeStruct((_NBLK * _CHUNK // 2, 128), jnp.float32),
)


@functools.lru_cache(maxsize=None)
def _make_sc_gather():
    mesh = plsc.VectorSubcoreMesh(core_axis_name="c", subcore_axis_name="s")

    @functools.partial(
        pl.kernel,
        mesh=mesh,
        compiler_params=pltpu.CompilerParams(use_tc_tiling_on_sc=False,
                                             needs_layout_passes=False),
        out_type=[
            jax.ShapeDtypeStruct((_B * 8, 16), jnp.float32),
            jax.ShapeDtypeStruct((_B * 8, 16), jnp.float32),
        ],
        scratch_types=[
            pltpu.VMEM((_BPW,), jnp.int32),
            pltpu.VMEM((_BPW,), jnp.int32),
            pltpu.VMEM((8, 128), jnp.int32),
            pltpu.VMEM((8, 128), jnp.int32),
            pltpu.VMEM((_BPW,), jnp.int32),
            pltpu.VMEM((_BPW,), jnp.int32),
            pltpu.VMEM((_BPW * 8, 16), jnp.float32),
            pltpu.VMEM((_BPW * 8, 16), jnp.float32),
            pltpu.VMEM((_BPW, 16), jnp.float32),
            pltpu.VMEM((_BPW, 16), jnp.float32),
            pltpu.SemaphoreType.DMA,
            pltpu.SemaphoreType.DMA,
        ],
    )
    def _sc_gather(x1_hbm, x2_hbm, gt_hbm, lin_hbm,
                   r1_out, r2_out,
                   idx1_v, idx2_v, g1_v, g2_v, lrow1_v, lrow2_v,
                   rows1_v, rows2_v, lbuf1_v, lbuf2_v,
                   sem1, sem2):
        wid = lax.axis_index("s") * _NC + lax.axis_index("c")
        base = wid * _BPW
        pltpu.sync_copy(x1_hbm.at[pl.ds(base, _BPW)], idx1_v)
        pltpu.sync_copy(x2_hbm.at[pl.ds(base, _BPW)], idx2_v)
        lane = lax.iota(jnp.int32, 16)
        for i in range(_BPW // 16):
            sl = pl.ds(i * 16, 16)
            idx2_v[sl] = idx2_v[sl] + _OFF

        for idx_v, g_v, lrow_v, rows_v, lbuf_v, r_out, sem in (
            (idx1_v, g1_v, lrow1_v, rows1_v, lbuf1_v, r1_out, sem1),
            (idx2_v, g2_v, lrow2_v, rows2_v, lbuf2_v, r2_out, sem2),
        ):
            for j in range(8):
                sl = pl.ds(j * 16, 16)
                v = idx_v[sl]
                jb = jnp.right_shift(v, _LOG2C)
                pos = v & (_CHUNK - 1)
                g0 = (jnp.left_shift(jb, _LOG2C + 2)
                      + jnp.left_shift(pos & (_CHUNK // 2 - 1), 3)
                      + jnp.left_shift(jnp.right_shift(pos, _LOG2C - 1), 2))
                row = lane * 0 + j
                for k in range(4):
                    plsc.store_scatter(g_v, [row, lane * 8 + k], g0 + k)
                for k in (4, 5, 6, 7):
                    plsc.store_scatter(g_v, [row, lane * 8 + k], g0)
                lrow_v[sl] = jnp.right_shift(v, 4)
            cps = [
                pltpu.async_copy(gt_hbm.at[g_v.at[j]],
                                 rows_v.at[pl.ds(j * 128, 128)], sem)
                for j in range(8)
            ]
            cpl = pltpu.async_copy(lin_hbm.at[lrow_v], lbuf_v, sem)
            for cp in cps:
                cp.wait()
            cpl.wait()
            for j in range(8):
                sl = pl.ds(j * 16, 16)
                samp = lane + (j * 16)
                vals = plsc.load_gather(lbuf_v, [samp, idx_v[sl] & 15])
                plsc.store_scatter(rows_v, [samp * 8 + 4, lane * 0], vals)
            pltpu.sync_copy(rows_v, r_out.at[pl.ds(base * 8, _BPW * 8)])

    return _sc_gather


def _dense_body(r1_ref, r2_ref, lin_b_ref,
                w1_ref, b1_ref, g1_ref, be1_ref,
                w2_ref, b2_ref, g2_ref, be2_ref,
                w3_ref, b3_ref, out_ref):
    e1 = r1_ref[:, 0:_D]
    e2 = r2_ref[:, 0:_D]

    s = e1 + e2
    fm = 0.5 * jnp.sum(s * s - e1 * e1 - e2 * e2, axis=1, keepdims=True)

    lin = r1_ref[:, _D:_D + 1] + r2_ref[:, _D:_D + 1] + lin_b_ref[...]

    h = (
        jnp.dot(e1, w1_ref[0:_D, :], preferred_element_type=jnp.float32)
        + jnp.dot(e2, w1_ref[_D:2 * _D, :], preferred_element_type=jnp.float32)
        + b1_ref[...]
    )
    m = jnp.mean(h, axis=0, keepdims=True)
    hc = h - m
    v = jnp.mean(hc * hc, axis=0, keepdims=True)
    h = jnp.maximum(hc * lax.rsqrt(v + 1e-5) * g1_ref[...] + be1_ref[...], 0.0)

    h = jnp.dot(h, w2_ref[...], preferred_element_type=jnp.float32) + b2_ref[...]
    m = jnp.mean(h, axis=0, keepdims=True)
    hc = h - m
    v = jnp.mean(hc * hc, axis=0, keepdims=True)
    h = jnp.maximum(hc * lax.rsqrt(v + 1e-5) * g2_ref[...] + be2_ref[...], 0.0)

    o = jnp.dot(h, w3_ref[...], preferred_element_type=jnp.float32) + b3_ref[...]
    z = lin + fm + o
    out_ref[...] = 1.0 / (1.0 + jnp.exp(-z))


_dense = pl.pallas_call(
    _dense_body,
    out_shape=jax.ShapeDtypeStruct((_B, 1), jnp.float32),
)


def kernel(x1, x2, emb_table, lin_w, lin_b,
           W1, b1, g1, be1, W2, b2, g2, be2, W3, b3):
    gt = _transposer(emb_table.T)
    r1, r2 = _make_sc_gather()(x1, x2, gt.reshape(_GROWS, 16),
                               lin_w.reshape(-1, 16))
    out = _dense(
        r1.reshape(_B, 128), r2.reshape(_B, 128), lin_b.reshape(1, 1),
        W1, b1.reshape(1, -1), g1.reshape(1, -1), be1.reshape(1, -1),
        W2, b2.reshape(1, -1), g2.reshape(1, -1), be2.reshape(1, -1),
        W3, b3.reshape(1, 1),
    )
    return out.reshape(_B)
